# Initial kernel scaffold; baseline (speedup 1.0000x reference)
import jax
import jax.numpy as jnp
from jax.experimental import pallas as pl


def kernel(F_p_raw, F_a_raw, edge_index_s, edge_index_p, edge_index_a, params):
    raise NotImplementedError("write your pallas kernel here")


# speedup vs baseline: 1.0480x; 1.0480x over previous
import jax
import jax.numpy as jnp
from jax.experimental import pallas as pl

N = 100000
E = 1600000
D_P, D_A, D_FEAT, HEADS = 3, 11, 32, 4
HID = D_FEAT // HEADS


def _gat(x, edge_index, W, a_src, a_dst, bias):
    loop = jnp.arange(N, dtype=edge_index.dtype)
    src = jnp.concatenate([edge_index[0], loop])
    dst = jnp.concatenate([edge_index[1], loop])
    h = (x @ W).reshape(N, HEADS, HID)
    al_src = (h * a_src[None]).sum(-1)
    al_dst = (h * a_dst[None]).sum(-1)
    e = al_src[src] + al_dst[dst]
    e = jax.nn.leaky_relu(e, 0.2)
    ex = jnp.exp(e)
    den = jax.ops.segment_sum(ex, dst, num_segments=N)
    num = jax.ops.segment_sum(h[src] * ex[:, :, None], dst, num_segments=N)
    out = num / (den[:, :, None] + 1e-16)
    return out.reshape(N, HEADS * HID) + bias


def _mlp_body(h_ref, w1_ref, b1_ref, w2_ref, b2_ref, o_ref):
    h = h_ref[...]
    t = jnp.maximum(h @ w1_ref[...] + b1_ref[...][None, :], 0.0)
    o_ref[...] = t @ w2_ref[...] + b2_ref[...][None, :]


def _mlp(h_cat, W1, b1, W2, b2):
    BLK = 2000
    grid = (N // BLK,)
    return pl.pallas_call(
        _mlp_body,
        grid=grid,
        in_specs=[
            pl.BlockSpec((BLK, 3 * D_FEAT), lambda i: (i, 0)),
            pl.BlockSpec((3 * D_FEAT, D_FEAT), lambda i: (0, 0)),
            pl.BlockSpec((D_FEAT,), lambda i: (0,)),
            pl.BlockSpec((D_FEAT, D_FEAT), lambda i: (0, 0)),
            pl.BlockSpec((D_FEAT,), lambda i: (0,)),
        ],
        out_specs=pl.BlockSpec((BLK, D_FEAT), lambda i: (i, 0)),
        out_shape=jax.ShapeDtypeStruct((N, D_FEAT), jnp.float32),
    )(h_cat, W1, b1, W2, b2)


def kernel(F_p_raw, F_a_raw, edge_index_s, edge_index_p, edge_index_a, p):
    f_s = jnp.concatenate([F_p_raw, F_a_raw], axis=-1) @ p['Ws'] + p['bs']
    f_p = F_p_raw @ p['Wp'] + p['bp']
    f_a = F_a_raw @ p['Wa'] + p['ba']
    h_s = jax.nn.elu(_gat(f_s, edge_index_s, p['Wg_s'], p['asrc_s'], p['adst_s'], p['bg_s']))
    h_p = jax.nn.elu(_gat(f_p, edge_index_p, p['Wg_p'], p['asrc_p'], p['adst_p'], p['bg_p']))
    h_a = jax.nn.elu(_gat(f_a, edge_index_a, p['Wg_a'], p['asrc_a'], p['adst_a'], p['bg_a']))
    h = jnp.concatenate([h_s, h_p, h_a], axis=-1)
    return _mlp(h, p['Wf1'], p['bf1'], p['Wf2'], p['bf2'])
